# baseline (speedup 1.0000x reference)
import functools

import jax
import jax.numpy as jnp
from jax import lax
from jax.experimental import pallas as pl
from jax.experimental.pallas import tpu as pltpu
from jax.experimental.pallas import tpu_sc as plsc

N = 10000
E = 320000
D = 128
L = 3

NC = 2
NS = 16
NW = NC * NS
NP = 10240
RPW = NP // NS
ZBR = 32

EPW = E // NW
EPWP = 10240
PADE = EPWP - EPW
CK = 80
CH = EPWP // CK


def _sc_body(h_hbm, src_hbm, dst_hbm, agg_out, cnt_out, src_r, dst_r,
             rows_r, zb_v, ones_v, zbc_v, agg_sh, cnt_sh, sem_g):
    cid = lax.axis_index("c")
    sid = lax.axis_index("s")
    wid = cid * NS + sid

    def _zrow(r, _):
        for c in range(D // 16):
            zb_v[r, pl.ds(c * 16, 16)] = jnp.zeros((16,), jnp.float32)
        return 0
    lax.fori_loop(0, ZBR, _zrow, 0)

    def _zcnt(k, _):
        zbc_v[pl.ds(k * 16, 16)] = jnp.zeros((16,), jnp.float32)
        return 0
    lax.fori_loop(0, RPW // 16, _zcnt, 0)

    def _ofill(k, _):
        ones_v[pl.ds(k * 16, 16)] = jnp.ones((16,), jnp.float32)
        return 0
    lax.fori_loop(0, CK // 16, _ofill, 0)

    for k in range(RPW // ZBR):
        pltpu.sync_copy(zb_v, agg_sh.at[pl.ds(sid * RPW + k * ZBR, ZBR)])
    pltpu.sync_copy(zbc_v, cnt_sh.at[pl.ds(sid * RPW, RPW)])
    plsc.subcore_barrier()

    def _step(j, _):
        e0 = wid * EPWP + j * CK
        pltpu.sync_copy(src_hbm.at[pl.ds(e0, CK)], src_r)
        pltpu.sync_copy(dst_hbm.at[pl.ds(e0, CK)], dst_r)
        pltpu.async_copy(h_hbm.at[src_r], rows_r, sem_g).wait()
        pltpu.sync_copy(rows_r, agg_sh.at[dst_r], add=True)
        pltpu.sync_copy(ones_v, cnt_sh.at[dst_r], add=True)
        return 0
    lax.fori_loop(0, CH, _step, 0)

    plsc.subcore_barrier()
    pltpu.sync_copy(agg_sh.at[pl.ds(sid * RPW, RPW)],
                    agg_out.at[cid, pl.ds(sid * RPW, RPW)])
    pltpu.sync_copy(cnt_sh.at[pl.ds(sid * RPW, RPW)], zbc_v)
    pltpu.sync_copy(zbc_v, cnt_out.at[pl.ds(cid * NP + sid * RPW, RPW)])


def _make_sc_agg():
    mesh = plsc.VectorSubcoreMesh(core_axis_name="c", subcore_axis_name="s")
    out_type = (jax.ShapeDtypeStruct((NC, NP, D), jnp.float32),
                jax.ShapeDtypeStruct((NC * NP,), jnp.float32))
    scratch = [
        pltpu.VMEM((CK,), jnp.int32),
        pltpu.VMEM((CK,), jnp.int32),
        pltpu.VMEM((CK, D), jnp.float32),
        pltpu.VMEM((ZBR, D), jnp.float32),
        pltpu.VMEM((CK,), jnp.float32),
        pltpu.VMEM((RPW,), jnp.float32),
        pltpu.VMEM_SHARED((NP, D), jnp.float32),
        pltpu.VMEM_SHARED((NP,), jnp.float32),
        pltpu.SemaphoreType.DMA,
    ]
    return pl.kernel(_sc_body, out_type=out_type, mesh=mesh,
                     scratch_types=scratch)


_sc_agg = _make_sc_agg()


def _dense_body(agg2_ref, cnt2_ref, h_ref, Wl_ref, bl_ref, Wr_ref,
                gamma_ref, beta_ref, relu_ref, out_ref):
    agg = agg2_ref[0, :N] + agg2_ref[1, :N]
    cnt = cnt2_ref[0, :N] + cnt2_ref[1, :N]
    mean = agg / jnp.maximum(cnt[:, None], 1.0)
    h = h_ref[...]
    out = (jnp.dot(mean, Wl_ref[...], preferred_element_type=jnp.float32)
           + bl_ref[...][None, :]
           + jnp.dot(h, Wr_ref[...], preferred_element_type=jnp.float32))
    nrm = jnp.sqrt(jnp.sum(out * out, axis=1, keepdims=True))
    out = out / jnp.maximum(nrm, 1e-12)
    mu = jnp.mean(out, axis=0, keepdims=True)
    var = jnp.mean((out - mu) * (out - mu), axis=0, keepdims=True)
    out = (gamma_ref[...][None, :] * (out - mu) / jnp.sqrt(var + 1e-5)
           + beta_ref[...][None, :] + h)
    out = jnp.where(relu_ref[0] > 0.0, jnp.maximum(out, 0.0), out)
    out_ref[...] = out


_dense = pl.pallas_call(
    _dense_body, out_shape=jax.ShapeDtypeStruct((N, D), jnp.float32))


def kernel(x, edge_index, Wl, bl, Wr, gamma, beta):
    src = jnp.concatenate(
        [edge_index[0].reshape(NW, EPW),
         jnp.zeros((NW, PADE), jnp.int32)], axis=1).reshape(-1)
    pad_dst = jnp.broadcast_to(N + jnp.arange(PADE, dtype=jnp.int32),
                               (NW, PADE))
    dst = jnp.concatenate(
        [edge_index[1].reshape(NW, EPW), pad_dst], axis=1).reshape(-1)
    relu_flags = jnp.arange(L, dtype=jnp.float32)[::-1].reshape(L, 1)

    def _layer(h, xs):
        Wl_i, bl_i, Wr_i, gamma_i, beta_i, relu_i = xs
        agg2, cnt2 = _sc_agg(h, src, dst)
        h = _dense(agg2, cnt2.reshape(NC, NP), h, Wl_i, bl_i, Wr_i,
                   gamma_i, beta_i, relu_i)
        return h, None

    h, _ = lax.scan(_layer, x,
                    (Wl[:L], bl[:L], Wr[:L], gamma[:L], beta[:L], relu_flags))
    return h


# speedup vs baseline: 2.1745x; 2.1745x over previous
import functools

import jax
import jax.numpy as jnp
from jax import lax
from jax.experimental import pallas as pl
from jax.experimental.pallas import tpu as pltpu
from jax.experimental.pallas import tpu_sc as plsc

N = 10000
E = 320000
D = 128
L = 3

NC = 2
NS = 16
NW = NC * NS
NP = 10240
RPW = NP // NS
ZBR = 32

EPW = E // NW
CK = 80
CH = EPW // CK


def _sc_body(h_hbm, src_hbm, dst_hbm, agg_out, cnt_out, src_r, dst_r,
             rows_r, zb_v, ones_v, zbc_v, agg_sh, cnt_sh, sem_g, sem_s):
    cid = lax.axis_index("c")
    sid = lax.axis_index("s")
    wid = cid * NS + sid

    def _zrow(r, _):
        for c in range(D // 16):
            zb_v[r, pl.ds(c * 16, 16)] = jnp.zeros((16,), jnp.float32)
        return 0
    lax.fori_loop(0, ZBR, _zrow, 0)

    def _zcnt(k, _):
        zbc_v[pl.ds(k * 16, 16)] = jnp.zeros((16,), jnp.float32)
        return 0
    lax.fori_loop(0, RPW // 16, _zcnt, 0)

    def _ofill(k, _):
        ones_v[pl.ds(k * 16, 16)] = jnp.ones((16,), jnp.float32)
        return 0
    lax.fori_loop(0, CK // 16, _ofill, 0)

    for k in range(RPW // ZBR):
        pltpu.sync_copy(zb_v, agg_sh.at[pl.ds(sid * RPW + k * ZBR, ZBR)])
    pltpu.sync_copy(zbc_v, cnt_sh.at[pl.ds(sid * RPW, RPW)])
    plsc.subcore_barrier()

    def _chunk(j, b, first):
        if not first:
            pltpu.make_async_copy(rows_r.at[b], agg_sh.at[dst_r.at[b]],
                                  sem_s.at[b]).wait()
            pltpu.make_async_copy(ones_v, cnt_sh.at[dst_r.at[b]],
                                  sem_s.at[b]).wait()
        e0 = wid * EPW + j * CK
        pltpu.sync_copy(src_hbm.at[pl.ds(e0, CK)], src_r.at[b])
        pltpu.sync_copy(dst_hbm.at[pl.ds(e0, CK)], dst_r.at[b])
        pltpu.async_copy(h_hbm.at[src_r.at[b]], rows_r.at[b], sem_g).wait()
        pltpu.async_copy(rows_r.at[b], agg_sh.at[dst_r.at[b]], sem_s.at[b],
                         add=True)
        pltpu.async_copy(ones_v, cnt_sh.at[dst_r.at[b]], sem_s.at[b],
                         add=True)

    _chunk(0, 0, True)
    _chunk(1, 1, True)

    def _group(g, _):
        j = 2 + g * 2
        _chunk(j, 0, False)
        _chunk(j + 1, 1, False)
        return 0
    lax.fori_loop(0, (CH - 3) // 2, _group, 0)
    _chunk(CH - 1, 0, False)
    for b in range(2):
        pltpu.make_async_copy(rows_r.at[b], agg_sh.at[dst_r.at[b]],
                              sem_s.at[b]).wait()
        pltpu.make_async_copy(ones_v, cnt_sh.at[dst_r.at[b]],
                              sem_s.at[b]).wait()

    plsc.subcore_barrier()
    pltpu.sync_copy(agg_sh.at[pl.ds(sid * RPW, RPW)],
                    agg_out.at[cid, pl.ds(sid * RPW, RPW)])
    pltpu.sync_copy(cnt_sh.at[pl.ds(sid * RPW, RPW)], zbc_v)
    pltpu.sync_copy(zbc_v, cnt_out.at[pl.ds(cid * NP + sid * RPW, RPW)])


def _make_sc_agg():
    mesh = plsc.VectorSubcoreMesh(core_axis_name="c", subcore_axis_name="s")
    out_type = (jax.ShapeDtypeStruct((NC, NP, D), jnp.float32),
                jax.ShapeDtypeStruct((NC * NP,), jnp.float32))
    scratch = [
        pltpu.VMEM((2, CK), jnp.int32),
        pltpu.VMEM((2, CK), jnp.int32),
        pltpu.VMEM((2, CK, D), jnp.float32),
        pltpu.VMEM((ZBR, D), jnp.float32),
        pltpu.VMEM((CK,), jnp.float32),
        pltpu.VMEM((RPW,), jnp.float32),
        pltpu.VMEM_SHARED((NP, D), jnp.float32),
        pltpu.VMEM_SHARED((NP,), jnp.float32),
        pltpu.SemaphoreType.DMA,
        pltpu.SemaphoreType.DMA((2,)),
    ]
    return pl.kernel(_sc_body, out_type=out_type, mesh=mesh,
                     scratch_types=scratch)


_sc_agg = _make_sc_agg()


def _dense_body(agg2_ref, cnt2_ref, h_ref, Wl_ref, bl_ref, Wr_ref,
                gamma_ref, beta_ref, relu_ref, out_ref):
    agg = agg2_ref[0, :N] + agg2_ref[1, :N]
    cnt = cnt2_ref[0, :N] + cnt2_ref[1, :N]
    mean = agg / jnp.maximum(cnt[:, None], 1.0)
    h = h_ref[...]
    out = (jnp.dot(mean, Wl_ref[...], preferred_element_type=jnp.float32)
           + bl_ref[...][None, :]
           + jnp.dot(h, Wr_ref[...], preferred_element_type=jnp.float32))
    nrm = jnp.sqrt(jnp.sum(out * out, axis=1, keepdims=True))
    out = out / jnp.maximum(nrm, 1e-12)
    mu = jnp.mean(out, axis=0, keepdims=True)
    var = jnp.mean((out - mu) * (out - mu), axis=0, keepdims=True)
    out = (gamma_ref[...][None, :] * (out - mu) / jnp.sqrt(var + 1e-5)
           + beta_ref[...][None, :] + h)
    out = jnp.where(relu_ref[0] > 0.0, jnp.maximum(out, 0.0), out)
    out_ref[...] = out


_dense = pl.pallas_call(
    _dense_body, out_shape=jax.ShapeDtypeStruct((N, D), jnp.float32))


def kernel(x, edge_index, Wl, bl, Wr, gamma, beta):
    src = edge_index[0]
    dst = edge_index[1]
    relu_flags = jnp.arange(L, dtype=jnp.float32)[::-1].reshape(L, 1)

    def _layer(h, xs):
        Wl_i, bl_i, Wr_i, gamma_i, beta_i, relu_i = xs
        agg2, cnt2 = _sc_agg(h, src, dst)
        h = _dense(agg2, cnt2.reshape(NC, NP), h, Wl_i, bl_i, Wr_i,
                   gamma_i, beta_i, relu_i)
        return h, None

    h, _ = lax.scan(_layer, x,
                    (Wl[:L], bl[:L], Wr[:L], gamma[:L], beta[:L], relu_flags))
    return h


# speedup vs baseline: 4.1626x; 1.9143x over previous
import functools

import jax
import jax.numpy as jnp
from jax import lax
from jax.experimental import pallas as pl
from jax.experimental.pallas import tpu as pltpu
from jax.experimental.pallas import tpu_sc as plsc

N = 10000
E = 320000
D = 128
L = 3

NC = 2
NS = 16
NW = NC * NS
NP = 10240
RPW = NP // NS
ZBR = 32

EPW = E // NW
CK = 80
CH = EPW // CK


def _sc_body(h_hbm, src_hbm, dst_hbm, agg_out, cnt_out, src_r, dst_r,
             rows_r, zb_v, ones_v, zbc_v, agg_sh, cnt_sh,
             sem_i, sem_g, sem_s):
    cid = lax.axis_index("c")
    sid = lax.axis_index("s")
    wid = cid * NS + sid

    def _zrow(r, _):
        for c in range(D // 16):
            zb_v[r, pl.ds(c * 16, 16)] = jnp.zeros((16,), jnp.float32)
        return 0
    lax.fori_loop(0, ZBR, _zrow, 0)

    def _zcnt(k, _):
        zbc_v[pl.ds(k * 16, 16)] = jnp.zeros((16,), jnp.float32)
        return 0
    lax.fori_loop(0, RPW // 16, _zcnt, 0)

    def _ofill(k, _):
        ones_v[pl.ds(k * 16, 16)] = jnp.ones((16,), jnp.float32)
        return 0
    lax.fori_loop(0, CK // 16, _ofill, 0)

    for k in range(RPW // ZBR):
        pltpu.sync_copy(zb_v, agg_sh.at[pl.ds(sid * RPW + k * ZBR, ZBR)])
    pltpu.sync_copy(zbc_v, cnt_sh.at[pl.ds(sid * RPW, RPW)])
    plsc.subcore_barrier()

    def _idx_load(jj, sync=False):
        s = jj % 4
        e0 = wid * EPW + jj * CK
        if sync:
            pltpu.sync_copy(src_hbm.at[pl.ds(e0, CK)], src_r.at[s])
            pltpu.sync_copy(dst_hbm.at[pl.ds(e0, CK)], dst_r.at[s])
        else:
            pltpu.async_copy(src_hbm.at[pl.ds(e0, CK)], src_r.at[s],
                             sem_i.at[s])
            pltpu.async_copy(dst_hbm.at[pl.ds(e0, CK)], dst_r.at[s],
                             sem_i.at[s])

    def _idx_wait(jj):
        s = jj % 4
        pltpu.make_async_copy(src_hbm.at[pl.ds(0, CK)], src_r.at[s],
                              sem_i.at[s]).wait()
        pltpu.make_async_copy(dst_hbm.at[pl.ds(0, CK)], dst_r.at[s],
                              sem_i.at[s]).wait()

    def _gather(jj):
        pltpu.async_copy(h_hbm.at[src_r.at[jj % 4]], rows_r.at[jj % 2],
                         sem_g.at[jj % 2])

    def _gather_wait(jj):
        pltpu.make_async_copy(h_hbm.at[src_r.at[jj % 4]],
                              rows_r.at[jj % 2], sem_g.at[jj % 2]).wait()

    def _scatter(jj):
        pltpu.async_copy(rows_r.at[jj % 2], agg_sh.at[dst_r.at[jj % 4]],
                         sem_s.at[jj % 2], add=True)
        pltpu.async_copy(ones_v, cnt_sh.at[dst_r.at[jj % 4]],
                         sem_s.at[jj % 2], add=True)

    def _scatter_wait(jj):
        pltpu.make_async_copy(rows_r.at[jj % 2], agg_sh.at[dst_r.at[jj % 4]],
                              sem_s.at[jj % 2]).wait()
        pltpu.make_async_copy(ones_v, cnt_sh.at[dst_r.at[jj % 4]],
                              sem_s.at[jj % 2]).wait()

    def _idx_load_t(jj_t, s):
        e0 = wid * EPW + jj_t * CK
        pltpu.async_copy(src_hbm.at[pl.ds(e0, CK)], src_r.at[s],
                         sem_i.at[s])
        pltpu.async_copy(dst_hbm.at[pl.ds(e0, CK)], dst_r.at[s],
                         sem_i.at[s])

    def _gather_t(jj_t, s, r):
        del jj_t
        pltpu.async_copy(h_hbm.at[src_r.at[s]], rows_r.at[r], sem_g.at[r])

    def _gather_wait_t(jj_t, s, r):
        del jj_t
        pltpu.make_async_copy(h_hbm.at[src_r.at[s]], rows_r.at[r],
                              sem_g.at[r]).wait()

    def _scatter_t(jj_t, s, r):
        del jj_t
        pltpu.async_copy(rows_r.at[r], agg_sh.at[dst_r.at[s]],
                         sem_s.at[r], add=True)
        pltpu.async_copy(ones_v, cnt_sh.at[dst_r.at[s]], sem_s.at[r],
                         add=True)

    _idx_load(0, sync=True)
    _idx_load(1, sync=True)
    _gather(0)
    _idx_load(2)
    _idx_load(3)
    _gather(1)
    _gather_wait(0)
    _scatter(0)
    _idx_wait(2)
    _scatter_wait(0)
    _idx_load(4)
    _gather(2)
    _gather_wait(1)
    _scatter(1)

    def _group(g, _):
        for t in range(4):
            j = 2 + t
            jt = 2 + g * 4 + t
            _idx_wait(j + 1)
            _scatter_wait(j - 1)
            _idx_load_t(jt + 3, (j + 3) % 4)
            _gather_t(jt + 1, (j + 1) % 4, (j + 1) % 2)
            _gather_wait_t(jt, j % 4, j % 2)
            _scatter_t(jt, j % 4, j % 2)
        return 0
    lax.fori_loop(0, (CH - 5) // 4, _group, 0)

    for j in (CH - 3, CH - 2):
        _idx_wait(j + 1)
        _scatter_wait(j - 1)
        _gather(j + 1)
        _gather_wait(j)
        _scatter(j)
    j = CH - 1
    _scatter_wait(j - 1)
    _gather_wait(j)
    _scatter(j)
    _scatter_wait(j)

    plsc.subcore_barrier()
    pltpu.sync_copy(agg_sh.at[pl.ds(sid * RPW, RPW)],
                    agg_out.at[cid, pl.ds(sid * RPW, RPW)])
    pltpu.sync_copy(cnt_sh.at[pl.ds(sid * RPW, RPW)], zbc_v)
    pltpu.sync_copy(zbc_v, cnt_out.at[pl.ds(cid * NP + sid * RPW, RPW)])


def _make_sc_agg():
    mesh = plsc.VectorSubcoreMesh(core_axis_name="c", subcore_axis_name="s")
    out_type = (jax.ShapeDtypeStruct((NC, NP, D), jnp.float32),
                jax.ShapeDtypeStruct((NC * NP,), jnp.float32))
    scratch = [
        pltpu.VMEM((4, CK), jnp.int32),
        pltpu.VMEM((4, CK), jnp.int32),
        pltpu.VMEM((2, CK, D), jnp.float32),
        pltpu.VMEM((ZBR, D), jnp.float32),
        pltpu.VMEM((CK,), jnp.float32),
        pltpu.VMEM((RPW,), jnp.float32),
        pltpu.VMEM_SHARED((NP, D), jnp.float32),
        pltpu.VMEM_SHARED((NP,), jnp.float32),
        pltpu.SemaphoreType.DMA((4,)),
        pltpu.SemaphoreType.DMA((2,)),
        pltpu.SemaphoreType.DMA((2,)),
    ]
    return pl.kernel(_sc_body, out_type=out_type, mesh=mesh,
                     scratch_types=scratch)


_sc_agg = _make_sc_agg()


def _dense_body(agg2_ref, cnt2_ref, h_ref, Wl_ref, bl_ref, Wr_ref,
                gamma_ref, beta_ref, relu_ref, out_ref):
    agg = agg2_ref[0, :N] + agg2_ref[1, :N]
    cnt = cnt2_ref[0, :N] + cnt2_ref[1, :N]
    mean = agg / jnp.maximum(cnt[:, None], 1.0)
    h = h_ref[...]
    out = (jnp.dot(mean, Wl_ref[...], preferred_element_type=jnp.float32)
           + bl_ref[...][None, :]
           + jnp.dot(h, Wr_ref[...], preferred_element_type=jnp.float32))
    nrm = jnp.sqrt(jnp.sum(out * out, axis=1, keepdims=True))
    out = out / jnp.maximum(nrm, 1e-12)
    mu = jnp.mean(out, axis=0, keepdims=True)
    var = jnp.mean((out - mu) * (out - mu), axis=0, keepdims=True)
    out = (gamma_ref[...][None, :] * (out - mu) / jnp.sqrt(var + 1e-5)
           + beta_ref[...][None, :] + h)
    out = jnp.where(relu_ref[0] > 0.0, jnp.maximum(out, 0.0), out)
    out_ref[...] = out


_dense = pl.pallas_call(
    _dense_body, out_shape=jax.ShapeDtypeStruct((N, D), jnp.float32))


def kernel(x, edge_index, Wl, bl, Wr, gamma, beta):
    src = edge_index[0]
    dst = edge_index[1]
    relu_flags = jnp.arange(L, dtype=jnp.float32)[::-1].reshape(L, 1)

    def _layer(h, xs):
        Wl_i, bl_i, Wr_i, gamma_i, beta_i, relu_i = xs
        agg2, cnt2 = _sc_agg(h, src, dst)
        h = _dense(agg2, cnt2.reshape(NC, NP), h, Wl_i, bl_i, Wr_i,
                   gamma_i, beta_i, relu_i)
        return h, None

    h, _ = lax.scan(_layer, x,
                    (Wl[:L], bl[:L], Wr[:L], gamma[:L], beta[:L], relu_flags))
    return h


# speedup vs baseline: 4.8452x; 1.1640x over previous
import functools

import jax
import jax.numpy as jnp
from jax import lax
from jax.experimental import pallas as pl
from jax.experimental.pallas import tpu as pltpu
from jax.experimental.pallas import tpu_sc as plsc

N = 10000
E = 320000
D = 128
L = 3

NC = 2
NS = 16
NW = NC * NS
NP = 10240
RPW = NP // NS
ZBR = 32

EPW = E // NW
CK = 80
CH = EPW // CK


def _sc_body(h_hbm, src_hbm, dst_hbm, agg_out, cnt_out, src_r, dst_r,
             rows_r, zb_v, ones_v, zbc_v, agg_sh, cnt_sh,
             sem_i, sem_g, sem_s):
    cid = lax.axis_index("c")
    sid = lax.axis_index("s")
    wid = cid * NS + sid

    def _zrow(r, _):
        for c in range(D // 16):
            zb_v[r, pl.ds(c * 16, 16)] = jnp.zeros((16,), jnp.float32)
        return 0
    lax.fori_loop(0, ZBR, _zrow, 0)

    def _zcnt(k, _):
        zbc_v[pl.ds(k * 16, 16)] = jnp.zeros((16,), jnp.float32)
        return 0
    lax.fori_loop(0, RPW // 16, _zcnt, 0)

    def _ofill(k, _):
        ones_v[pl.ds(k * 16, 16)] = jnp.ones((16,), jnp.float32)
        return 0
    lax.fori_loop(0, CK // 16, _ofill, 0)

    for k in range(RPW // ZBR):
        pltpu.sync_copy(zb_v, agg_sh.at[pl.ds(sid * RPW + k * ZBR, ZBR)])
    pltpu.sync_copy(zbc_v, cnt_sh.at[pl.ds(sid * RPW, RPW)])
    plsc.subcore_barrier()

    def _idx_load(jj, sync=False):
        s = jj % 4
        e0 = wid * EPW + jj * CK
        if sync:
            pltpu.sync_copy(src_hbm.at[pl.ds(e0, CK)], src_r.at[s])
            pltpu.sync_copy(dst_hbm.at[pl.ds(e0, CK)], dst_r.at[s])
        else:
            pltpu.async_copy(src_hbm.at[pl.ds(e0, CK)], src_r.at[s],
                             sem_i.at[s])
            pltpu.async_copy(dst_hbm.at[pl.ds(e0, CK)], dst_r.at[s],
                             sem_i.at[s])

    def _idx_wait(jj):
        s = jj % 4
        pltpu.make_async_copy(src_hbm.at[pl.ds(0, CK)], src_r.at[s],
                              sem_i.at[s]).wait()
        pltpu.make_async_copy(dst_hbm.at[pl.ds(0, CK)], dst_r.at[s],
                              sem_i.at[s]).wait()

    def _gather(jj):
        pltpu.async_copy(h_hbm.at[src_r.at[jj % 4]], rows_r.at[jj % 3],
                         sem_g.at[jj % 3])

    def _gather_wait(jj):
        pltpu.make_async_copy(h_hbm.at[src_r.at[jj % 4]],
                              rows_r.at[jj % 3], sem_g.at[jj % 3]).wait()

    def _scatter(jj):
        pltpu.async_copy(rows_r.at[jj % 3], agg_sh.at[dst_r.at[jj % 4]],
                         sem_s.at[jj % 3], add=True)
        pltpu.async_copy(ones_v, cnt_sh.at[dst_r.at[jj % 4]],
                         sem_s.at[jj % 3], add=True)

    def _scatter_wait(jj):
        pltpu.make_async_copy(rows_r.at[jj % 3], agg_sh.at[dst_r.at[jj % 4]],
                              sem_s.at[jj % 3]).wait()
        pltpu.make_async_copy(ones_v, cnt_sh.at[dst_r.at[jj % 4]],
                              sem_s.at[jj % 3]).wait()

    def _idx_load_t(jj_t, s):
        e0 = wid * EPW + jj_t * CK
        pltpu.async_copy(src_hbm.at[pl.ds(e0, CK)], src_r.at[s],
                         sem_i.at[s])
        pltpu.async_copy(dst_hbm.at[pl.ds(e0, CK)], dst_r.at[s],
                         sem_i.at[s])

    def _gather_t(jj_t, s, r):
        del jj_t
        pltpu.async_copy(h_hbm.at[src_r.at[s]], rows_r.at[r], sem_g.at[r])

    def _gather_wait_t(jj_t, s, r):
        del jj_t
        pltpu.make_async_copy(h_hbm.at[src_r.at[s]], rows_r.at[r],
                              sem_g.at[r]).wait()

    def _scatter_t(jj_t, s, r):
        del jj_t
        pltpu.async_copy(rows_r.at[r], agg_sh.at[dst_r.at[s]],
                         sem_s.at[r], add=True)
        pltpu.async_copy(ones_v, cnt_sh.at[dst_r.at[s]], sem_s.at[r],
                         add=True)

    _idx_load(0, sync=True)
    _idx_load(1, sync=True)
    _gather(0)
    _gather(1)
    _idx_load(2)
    _idx_wait(2)
    _idx_load(3)
    _gather(2)
    _gather_wait(0)
    _scatter(0)
    _idx_wait(3)
    _scatter_wait(0)
    _idx_load(4)
    _gather(3)
    _gather_wait(1)
    _scatter(1)

    def _group(g, _):
        for t in range(12):
            j = 2 + t
            jt = 2 + g * 12 + t
            _idx_wait(j + 2)
            _scatter_wait(j - 1)
            _idx_load_t(jt + 3, (j + 3) % 4)
            _gather_t(jt + 2, (j + 2) % 4, (j + 2) % 3)
            _gather_wait_t(jt, j % 4, j % 3)
            _scatter_t(jt, j % 4, j % 3)
        return 0
    lax.fori_loop(0, (CH - 5) // 12, _group, 0)

    j = CH - 3
    _idx_wait(j + 2)
    _scatter_wait(j - 1)
    _gather(j + 2)
    _gather_wait(j)
    _scatter(j)
    for j in (CH - 2, CH - 1):
        _scatter_wait(j - 1)
        _gather_wait(j)
        _scatter(j)
    _scatter_wait(CH - 1)

    plsc.subcore_barrier()
    pltpu.sync_copy(agg_sh.at[pl.ds(sid * RPW, RPW)],
                    agg_out.at[cid, pl.ds(sid * RPW, RPW)])
    pltpu.sync_copy(cnt_sh.at[pl.ds(sid * RPW, RPW)], zbc_v)
    pltpu.sync_copy(zbc_v, cnt_out.at[pl.ds(cid * NP + sid * RPW, RPW)])


def _make_sc_agg():
    mesh = plsc.VectorSubcoreMesh(core_axis_name="c", subcore_axis_name="s")
    out_type = (jax.ShapeDtypeStruct((NC, NP, D), jnp.float32),
                jax.ShapeDtypeStruct((NC * NP,), jnp.float32))
    scratch = [
        pltpu.VMEM((4, CK), jnp.int32),
        pltpu.VMEM((4, CK), jnp.int32),
        pltpu.VMEM((3, CK, D), jnp.float32),
        pltpu.VMEM((ZBR, D), jnp.float32),
        pltpu.VMEM((CK,), jnp.float32),
        pltpu.VMEM((RPW,), jnp.float32),
        pltpu.VMEM_SHARED((NP, D), jnp.float32),
        pltpu.VMEM_SHARED((NP,), jnp.float32),
        pltpu.SemaphoreType.DMA((4,)),
        pltpu.SemaphoreType.DMA((3,)),
        pltpu.SemaphoreType.DMA((3,)),
    ]
    return pl.kernel(_sc_body, out_type=out_type, mesh=mesh,
                     scratch_types=scratch)


_sc_agg = _make_sc_agg()


def _dense_body(agg2_ref, cnt2_ref, h_ref, Wl_ref, bl_ref, Wr_ref,
                gamma_ref, beta_ref, relu_ref, out_ref):
    agg = agg2_ref[0, :N] + agg2_ref[1, :N]
    cnt = cnt2_ref[0, :N] + cnt2_ref[1, :N]
    mean = agg / jnp.maximum(cnt[:, None], 1.0)
    h = h_ref[...]
    out = (jnp.dot(mean, Wl_ref[...], preferred_element_type=jnp.float32)
           + bl_ref[...][None, :]
           + jnp.dot(h, Wr_ref[...], preferred_element_type=jnp.float32))
    nrm = jnp.sqrt(jnp.sum(out * out, axis=1, keepdims=True))
    out = out / jnp.maximum(nrm, 1e-12)
    mu = jnp.mean(out, axis=0, keepdims=True)
    var = jnp.mean((out - mu) * (out - mu), axis=0, keepdims=True)
    out = (gamma_ref[...][None, :] * (out - mu) / jnp.sqrt(var + 1e-5)
           + beta_ref[...][None, :] + h)
    out = jnp.where(relu_ref[0] > 0.0, jnp.maximum(out, 0.0), out)
    out_ref[...] = out


_dense = pl.pallas_call(
    _dense_body, out_shape=jax.ShapeDtypeStruct((N, D), jnp.float32))


def kernel(x, edge_index, Wl, bl, Wr, gamma, beta):
    src = edge_index[0]
    dst = edge_index[1]
    relu_flags = jnp.arange(L, dtype=jnp.float32)[::-1].reshape(L, 1)

    def _layer(h, xs):
        Wl_i, bl_i, Wr_i, gamma_i, beta_i, relu_i = xs
        agg2, cnt2 = _sc_agg(h, src, dst)
        h = _dense(agg2, cnt2.reshape(NC, NP), h, Wl_i, bl_i, Wr_i,
                   gamma_i, beta_i, relu_i)
        return h, None

    h, _ = lax.scan(_layer, x,
                    (Wl[:L], bl[:L], Wr[:L], gamma[:L], beta[:L], relu_flags))
    return h


# speedup vs baseline: 4.8500x; 1.0010x over previous
import functools

import jax
import jax.numpy as jnp
from jax import lax
from jax.experimental import pallas as pl
from jax.experimental.pallas import tpu as pltpu
from jax.experimental.pallas import tpu_sc as plsc

N = 10000
E = 320000
D = 128
L = 3

NC = 2
NS = 16
NW = NC * NS
NP = 10240
RPW = NP // NS
ZBR = 32

EPW = E // NW
CK = 80
CH = EPW // CK


def _sc_body(h_hbm, src_hbm, dst_hbm, flag_hbm, agg_out, cnt_out,
             src_r, dst_r, rows_r, zb_v, ones_v, zbc_v, flag_v,
             agg_sh, cnt_sh, sem_i, sem_g, sem_s, sem_z):
    cid = lax.axis_index("c")
    sid = lax.axis_index("s")
    wid = cid * NS + sid

    pltpu.sync_copy(flag_hbm, flag_v)
    do_cnt = jnp.any(flag_v[pl.ds(0, 16)] > 0)

    def _zrow(r, _):
        for c in range(D // 16):
            zb_v[r, pl.ds(c * 16, 16)] = jnp.zeros((16,), jnp.float32)
        return 0
    lax.fori_loop(0, ZBR, _zrow, 0)

    @pl.when(do_cnt)
    def _cnt_init():
        def _zcnt(k, _):
            zbc_v[pl.ds(k * 16, 16)] = jnp.zeros((16,), jnp.float32)
            return 0
        lax.fori_loop(0, RPW // 16, _zcnt, 0)

        def _ofill(k, _):
            ones_v[pl.ds(k * 16, 16)] = jnp.ones((16,), jnp.float32)
            return 0
        lax.fori_loop(0, CK // 16, _ofill, 0)
        pltpu.sync_copy(zbc_v, cnt_sh.at[pl.ds(sid * RPW, RPW)])

    for k in range(RPW // ZBR):
        pltpu.async_copy(zb_v, agg_sh.at[pl.ds(sid * RPW + k * ZBR, ZBR)],
                         sem_z)
    for k in range(RPW // ZBR):
        pltpu.make_async_copy(
            zb_v, agg_sh.at[pl.ds(sid * RPW + k * ZBR, ZBR)], sem_z).wait()
    plsc.subcore_barrier()

    def _idx_load(jj, sync=False):
        s = jj % 4
        e0 = wid * EPW + jj * CK
        if sync:
            pltpu.sync_copy(src_hbm.at[pl.ds(e0, CK)], src_r.at[s])
            pltpu.sync_copy(dst_hbm.at[pl.ds(e0, CK)], dst_r.at[s])
        else:
            pltpu.async_copy(src_hbm.at[pl.ds(e0, CK)], src_r.at[s],
                             sem_i.at[s])
            pltpu.async_copy(dst_hbm.at[pl.ds(e0, CK)], dst_r.at[s],
                             sem_i.at[s])

    def _idx_wait(jj):
        s = jj % 4
        pltpu.make_async_copy(src_hbm.at[pl.ds(0, CK)], src_r.at[s],
                              sem_i.at[s]).wait()
        pltpu.make_async_copy(dst_hbm.at[pl.ds(0, CK)], dst_r.at[s],
                              sem_i.at[s]).wait()

    def _gather(jj):
        pltpu.async_copy(h_hbm.at[src_r.at[jj % 4]], rows_r.at[jj % 3],
                         sem_g.at[jj % 3])

    def _gather_wait(jj):
        pltpu.make_async_copy(h_hbm.at[src_r.at[jj % 4]],
                              rows_r.at[jj % 3], sem_g.at[jj % 3]).wait()

    def _scatter(jj):
        pltpu.async_copy(rows_r.at[jj % 3], agg_sh.at[dst_r.at[jj % 4]],
                         sem_s.at[jj % 3], add=True)
        @pl.when(do_cnt)
        def _sc():
            pltpu.async_copy(ones_v, cnt_sh.at[dst_r.at[jj % 4]],
                             sem_s.at[jj % 3], add=True)

    def _scatter_wait(jj):
        pltpu.make_async_copy(rows_r.at[jj % 3], agg_sh.at[dst_r.at[jj % 4]],
                              sem_s.at[jj % 3]).wait()
        @pl.when(do_cnt)
        def _scw():
            pltpu.make_async_copy(ones_v, cnt_sh.at[dst_r.at[jj % 4]],
                                  sem_s.at[jj % 3]).wait()

    def _idx_load_t(jj_t, s):
        e0 = wid * EPW + jj_t * CK
        pltpu.async_copy(src_hbm.at[pl.ds(e0, CK)], src_r.at[s],
                         sem_i.at[s])
        pltpu.async_copy(dst_hbm.at[pl.ds(e0, CK)], dst_r.at[s],
                         sem_i.at[s])

    def _gather_t(jj_t, s, r):
        del jj_t
        pltpu.async_copy(h_hbm.at[src_r.at[s]], rows_r.at[r], sem_g.at[r])

    def _gather_wait_t(jj_t, s, r):
        del jj_t
        pltpu.make_async_copy(h_hbm.at[src_r.at[s]], rows_r.at[r],
                              sem_g.at[r]).wait()

    def _scatter_t(jj_t, s, r):
        del jj_t
        pltpu.async_copy(rows_r.at[r], agg_sh.at[dst_r.at[s]],
                         sem_s.at[r], add=True)
        @pl.when(do_cnt)
        def _sct():
            pltpu.async_copy(ones_v, cnt_sh.at[dst_r.at[s]], sem_s.at[r],
                             add=True)

    _idx_load(0, sync=True)
    _idx_load(1, sync=True)
    _gather(0)
    _gather(1)
    _idx_load(2)
    _idx_wait(2)
    _idx_load(3)
    _gather(2)
    _gather_wait(0)
    _scatter(0)
    _idx_wait(3)
    _scatter_wait(0)
    _idx_load(4)
    _gather(3)
    _gather_wait(1)
    _scatter(1)

    def _group(g, _):
        for t in range(12):
            j = 2 + t
            jt = 2 + g * 12 + t
            _idx_wait(j + 2)
            _scatter_wait(j - 1)
            _idx_load_t(jt + 3, (j + 3) % 4)
            _gather_t(jt + 2, (j + 2) % 4, (j + 2) % 3)
            _gather_wait_t(jt, j % 4, j % 3)
            _scatter_t(jt, j % 4, j % 3)
        return 0
    lax.fori_loop(0, (CH - 5) // 12, _group, 0)

    j = CH - 3
    _idx_wait(j + 2)
    _scatter_wait(j - 1)
    _gather(j + 2)
    _gather_wait(j)
    _scatter(j)
    for j in (CH - 2, CH - 1):
        _scatter_wait(j - 1)
        _gather_wait(j)
        _scatter(j)
    _scatter_wait(CH - 1)

    plsc.subcore_barrier()
    pltpu.sync_copy(agg_sh.at[pl.ds(sid * RPW, RPW)],
                    agg_out.at[cid, pl.ds(sid * RPW, RPW)])
    @pl.when(do_cnt)
    def _cnt_out():
        pltpu.sync_copy(cnt_sh.at[pl.ds(sid * RPW, RPW)], zbc_v)
        pltpu.sync_copy(zbc_v, cnt_out.at[pl.ds(cid * NP + sid * RPW, RPW)])


def _make_sc_agg():
    mesh = plsc.VectorSubcoreMesh(core_axis_name="c", subcore_axis_name="s")
    out_type = (jax.ShapeDtypeStruct((NC, NP, D), jnp.float32),
                jax.ShapeDtypeStruct((NC * NP,), jnp.float32))
    scratch = [
        pltpu.VMEM((4, CK), jnp.int32),
        pltpu.VMEM((4, CK), jnp.int32),
        pltpu.VMEM((3, CK, D), jnp.float32),
        pltpu.VMEM((ZBR, D), jnp.float32),
        pltpu.VMEM((CK,), jnp.float32),
        pltpu.VMEM((RPW,), jnp.float32),
        pltpu.VMEM((16,), jnp.int32),
        pltpu.VMEM_SHARED((NP, D), jnp.float32),
        pltpu.VMEM_SHARED((NP,), jnp.float32),
        pltpu.SemaphoreType.DMA((4,)),
        pltpu.SemaphoreType.DMA((3,)),
        pltpu.SemaphoreType.DMA((3,)),
        pltpu.SemaphoreType.DMA,
    ]
    return pl.kernel(_sc_body, out_type=out_type, mesh=mesh,
                     scratch_types=scratch,
                     compiler_params=pltpu.CompilerParams(
                         needs_layout_passes=False))


_sc_agg = _make_sc_agg()


def _dense_body(agg2_ref, cnt2_ref, h_ref, Wl_ref, bl_ref, Wr_ref,
                gamma_ref, beta_ref, relu_ref, out_ref):
    agg = agg2_ref[0, :N] + agg2_ref[1, :N]
    cnt = cnt2_ref[0, :N] + cnt2_ref[1, :N]
    mean = agg / jnp.maximum(cnt[:, None], 1.0)
    h = h_ref[...]
    out = (jnp.dot(mean, Wl_ref[...], preferred_element_type=jnp.float32)
           + bl_ref[...][None, :]
           + jnp.dot(h, Wr_ref[...], preferred_element_type=jnp.float32))
    nrm = jnp.sqrt(jnp.sum(out * out, axis=1, keepdims=True))
    out = out / jnp.maximum(nrm, 1e-12)
    mu = jnp.mean(out, axis=0, keepdims=True)
    var = jnp.mean((out - mu) * (out - mu), axis=0, keepdims=True)
    out = (gamma_ref[...][None, :] * (out - mu) / jnp.sqrt(var + 1e-5)
           + beta_ref[...][None, :] + h)
    out = jnp.where(relu_ref[0] > 0.0, jnp.maximum(out, 0.0), out)
    out_ref[...] = out


_dense = pl.pallas_call(
    _dense_body, out_shape=jax.ShapeDtypeStruct((N, D), jnp.float32))


def kernel(x, edge_index, Wl, bl, Wr, gamma, beta):
    src = edge_index[0]
    dst = edge_index[1]
    relu_flags = jnp.arange(L, dtype=jnp.float32)[::-1].reshape(L, 1)

    cnt_flags = jnp.zeros((L, 16), jnp.int32).at[0].set(1)

    def _layer(carry, xs):
        h, cnt2 = carry
        Wl_i, bl_i, Wr_i, gamma_i, beta_i, relu_i, flag_i = xs
        agg2, cnt2_new = _sc_agg(h, src, dst, flag_i)
        cnt2 = jnp.where(flag_i[0] > 0, cnt2_new.reshape(NC, NP), cnt2)
        h = _dense(agg2, cnt2, h, Wl_i, bl_i, Wr_i,
                   gamma_i, beta_i, relu_i)
        return (h, cnt2), None

    (h, _), _ = lax.scan(
        _layer, (x, jnp.zeros((NC, NP), jnp.float32)),
        (Wl[:L], bl[:L], Wr[:L], gamma[:L], beta[:L], relu_flags,
         cnt_flags))
    return h
